# baseline (speedup 1.0000x reference)
import functools

import jax
import jax.numpy as jnp
from jax import lax
from jax.experimental import pallas as pl
from jax.experimental.pallas import tpu as pltpu
from jax.experimental.pallas import tpu_sc as plsc

NC = 2
NS = 16
NW = NC * NS

NB = 4096
T = 200
D = 64
V = 8193

BROWS = NB // NW
CB = 2
CH = 100
K = CB * T // CH
NCHUNK = BROWS // CB
IDXR = BROWS * T // CH


def _body(idx_hbm, table_hbm, out_hbm, table_sh, idx_v, rows_v,
          gs0, gs1, ws0, ws1):
    sid = lax.axis_index("s")
    wid = sid * NC + lax.axis_index("c")
    base = wid * BROWS
    gsem = (gs0, gs1)
    wsem = (ws0, ws1)

    @pl.when(sid == 0)
    def _():
        pltpu.sync_copy(table_hbm, table_sh)

    pltpu.sync_copy(idx_hbm.at[wid], idx_v)
    plsc.subcore_barrier()

    def fire_gathers(t, s):
        for j in range(K):
            pltpu.async_copy(
                table_sh.at[idx_v.at[t * K + j]],
                rows_v.at[s].at[j // 2].at[pl.ds((j % 2) * CH, CH)],
                gsem[s],
            )

    def drain_gathers(t, s):
        for j in range(K):
            pltpu.make_async_copy(
                table_sh.at[idx_v.at[t * K + j]],
                rows_v.at[s].at[j // 2].at[pl.ds((j % 2) * CH, CH)],
                gsem[s],
            ).wait()

    def fire_write(t, s):
        pltpu.async_copy(
            rows_v.at[s],
            out_hbm.at[pl.ds(base + t * CB, CB)],
            wsem[s],
        )

    def wait_write(t, s):
        pltpu.make_async_copy(
            rows_v.at[s],
            out_hbm.at[pl.ds(base + t * CB, CB)],
            wsem[s],
        ).wait()

    fire_gathers(0, 0)
    fire_gathers(1, 1)
    drain_gathers(0, 0)
    fire_write(0, 0)

    def step(i, carry):
        t = 2 * i
        wait_write(t - 2, 0)
        fire_gathers(t, 0)
        drain_gathers(t - 1, 1)
        fire_write(t - 1, 1)
        wait_write(t - 1, 1)
        fire_gathers(t + 1, 1)
        drain_gathers(t, 0)
        fire_write(t, 0)
        return carry

    lax.fori_loop(1, NCHUNK // 2, step, 0)

    drain_gathers(NCHUNK - 1, 1)
    fire_write(NCHUNK - 1, 1)
    wait_write(NCHUNK - 2, 0)
    wait_write(NCHUNK - 1, 1)


@functools.partial(jax.jit, static_argnums=())
def kernel(token_indices, position_embedding_matrix):
    idx = token_indices.astype(jnp.int32).reshape(NW, IDXR, CH)
    run = pl.kernel(
        _body,
        out_type=jax.ShapeDtypeStruct((NB, T, D), jnp.float32),
        mesh=plsc.VectorSubcoreMesh(core_axis_name="c", subcore_axis_name="s"),
        scratch_types=[
            pltpu.VMEM_SHARED((V, D), jnp.float32),
            pltpu.VMEM((IDXR, CH), jnp.int32),
            pltpu.VMEM((2, CB, T, D), jnp.float32),
            pltpu.SemaphoreType.DMA,
            pltpu.SemaphoreType.DMA,
            pltpu.SemaphoreType.DMA,
            pltpu.SemaphoreType.DMA,
        ],
        compiler_params=pltpu.CompilerParams(use_tc_tiling_on_sc=False),
    )
    return run(idx, position_embedding_matrix)
